# baseline (speedup 1.0000x reference)
import functools

import jax
import jax.numpy as jnp
from jax import lax
from jax.experimental import pallas as pl
from jax.experimental.pallas import tpu as pltpu
from jax.experimental.pallas import tpu_sc as plsc

B, S, D = 4, 2048, 1024
NC, NS = 2, 16
NW = NC * NS
S_PER_W = S // NW
R = 32
LANES = 16


def _body(x_hbm, pt_hbm, out_hbm, pe_buf, x_buf):
    wid = lax.axis_index("s") * NC + lax.axis_index("c")
    s0 = wid * S_PER_W
    pltpu.sync_copy(pt_hbm.at[pl.ds(s0, S_PER_W)], pe_buf)

    for b in range(B):
        for sub in range(S_PER_W // R):
            row0 = s0 + sub * R
            pltpu.sync_copy(x_hbm.at[b, pl.ds(row0, R)], x_buf)

            def add_row(r, _):
                def add_vec(c, _):
                    sl = pl.ds(c * LANES, LANES)
                    x_buf[r, sl] = x_buf[r, sl] + pe_buf[sub * R + r, sl]
                    return 0

                return lax.fori_loop(0, D // LANES, add_vec, 0)

            lax.fori_loop(0, R, add_row, 0)
            pltpu.sync_copy(x_buf, out_hbm.at[b, pl.ds(row0, R)])


@jax.jit
def kernel(x, pos_table):
    mesh = plsc.VectorSubcoreMesh(core_axis_name="c", subcore_axis_name="s")
    return pl.kernel(
        _body,
        out_type=jax.ShapeDtypeStruct((B, S, D), jnp.float32),
        mesh=mesh,
        scratch_types=[
            pltpu.VMEM((S_PER_W, D), jnp.float32),
            pltpu.VMEM((R, D), jnp.float32),
        ],
    )(x, pos_table)


# speedup vs baseline: 1.0211x; 1.0211x over previous
import jax
import jax.numpy as jnp
from jax import lax
from jax.experimental import pallas as pl
from jax.experimental.pallas import tpu as pltpu
from jax.experimental.pallas import tpu_sc as plsc

B, S, D = 4, 2048, 1024
NC, NS = 2, 16
NW = NC * NS
S_PER_W = S // NW
R = 16
CHUNK = R * D
SUBS = S_PER_W // R
NCHUNKS = B * SUBS
LANES = 16


def _body(x_hbm, pt_hbm, out_hbm, pe_buf, xb0, xb1, si0, si1, so0, so1):
    wid = lax.axis_index("s") * NC + lax.axis_index("c")
    s0 = wid * S_PER_W
    bufs = (xb0, xb1)
    in_sems = (si0, si1)
    out_sems = (so0, so1)

    pltpu.sync_copy(pt_hbm.at[pl.ds(s0 * D, S_PER_W * D)], pe_buf)

    def hbm_off(i):
        b, sub = divmod(i, SUBS)
        return (b * S + sub * R) * D + s0 * D

    def in_copy(i):
        return pltpu.make_async_copy(
            x_hbm.at[pl.ds(hbm_off(i), CHUNK)], bufs[i % 2], in_sems[i % 2])

    def out_copy(i):
        return pltpu.make_async_copy(
            bufs[i % 2], out_hbm.at[pl.ds(hbm_off(i), CHUNK)], out_sems[i % 2])

    in_copy(0).start()
    for i in range(NCHUNKS):
        nb = i % 2
        if i + 1 < NCHUNKS:
            if i >= 1:
                out_copy(i - 1).wait()
            in_copy(i + 1).start()
        in_copy(i).wait()

        pe_base = (i % SUBS) * CHUNK
        buf = bufs[nb]

        @plsc.parallel_loop(0, CHUNK // LANES, unroll=8)
        def _(v):
            off = v * LANES
            plsc.addupdate(
                buf.at[pl.ds(off, LANES)],
                pe_buf[pl.ds(pe_base + off, LANES)],
            )

        out_copy(i).start()
    out_copy(NCHUNKS - 2).wait()
    out_copy(NCHUNKS - 1).wait()


@jax.jit
def kernel(x, pos_table):
    mesh = plsc.VectorSubcoreMesh(core_axis_name="c", subcore_axis_name="s")
    out = pl.kernel(
        _body,
        out_type=jax.ShapeDtypeStruct((B * S * D,), jnp.float32),
        mesh=mesh,
        scratch_types=[
            pltpu.VMEM((S_PER_W * D,), jnp.float32),
            pltpu.VMEM((CHUNK,), jnp.float32),
            pltpu.VMEM((CHUNK,), jnp.float32),
            pltpu.SemaphoreType.DMA,
            pltpu.SemaphoreType.DMA,
            pltpu.SemaphoreType.DMA,
            pltpu.SemaphoreType.DMA,
        ],
    )(x.reshape(-1), pos_table.reshape(-1))
    return out.reshape(B, S, D)


# speedup vs baseline: 2.3817x; 2.3324x over previous
import jax
import jax.numpy as jnp
from jax import lax
from jax.experimental import pallas as pl
from jax.experimental.pallas import tpu as pltpu
from jax.experimental.pallas import tpu_sc as plsc

B, S, D = 4, 2048, 1024
NC, NS = 2, 16
NW = NC * NS
S_PER_W = S // NW
R = 16
SUBS = S_PER_W // R
NCHUNKS = B * SUBS
LANES = 16
VREGS_PER_ROW = D // LANES
VREGS_PER_CHUNK = R * VREGS_PER_ROW


def _body(x_hbm, pt_hbm, out_hbm, pe_buf, xb0, xb1, si0, si1, so0, so1):
    wid = lax.axis_index("s") * NC + lax.axis_index("c")
    s0 = wid * S_PER_W
    bufs = (xb0, xb1)
    in_sems = (si0, si1)
    out_sems = (so0, so1)

    pltpu.sync_copy(pt_hbm.at[pl.ds(s0, S_PER_W)], pe_buf)

    def in_copy(i):
        b, sub = divmod(i, SUBS)
        return pltpu.make_async_copy(
            x_hbm.at[b, pl.ds(s0 + sub * R, R)], bufs[i % 2], in_sems[i % 2])

    def out_copy(i):
        b, sub = divmod(i, SUBS)
        return pltpu.make_async_copy(
            bufs[i % 2], out_hbm.at[b, pl.ds(s0 + sub * R, R)],
            out_sems[i % 2])

    in_copy(0).start()
    for i in range(NCHUNKS):
        if i + 1 < NCHUNKS:
            if i >= 1:
                out_copy(i - 1).wait()
            in_copy(i + 1).start()
        in_copy(i).wait()

        buf = bufs[i % 2]
        row_base = (i % SUBS) * R

        @plsc.parallel_loop(0, VREGS_PER_CHUNK, unroll=8)
        def _(v):
            r = v >> 6
            coff = (v & (VREGS_PER_ROW - 1)) * LANES
            plsc.addupdate(
                buf.at[r, pl.ds(coff, LANES)],
                pe_buf[row_base + r, pl.ds(coff, LANES)],
            )

        out_copy(i).start()
    out_copy(NCHUNKS - 2).wait()
    out_copy(NCHUNKS - 1).wait()


@jax.jit
def kernel(x, pos_table):
    mesh = plsc.VectorSubcoreMesh(core_axis_name="c", subcore_axis_name="s")
    return pl.kernel(
        _body,
        out_type=jax.ShapeDtypeStruct((B, S, D), jnp.float32),
        mesh=mesh,
        scratch_types=[
            pltpu.VMEM((S_PER_W, D), jnp.float32),
            pltpu.VMEM((R, D), jnp.float32),
            pltpu.VMEM((R, D), jnp.float32),
            pltpu.SemaphoreType.DMA,
            pltpu.SemaphoreType.DMA,
            pltpu.SemaphoreType.DMA,
            pltpu.SemaphoreType.DMA,
        ],
    )(x, pos_table)
